# baseline (speedup 1.0000x reference)
import functools

import jax
import jax.numpy as jnp
from jax.experimental import pallas as pl
from jax.experimental.pallas import tpu as pltpu

_U_CHUNKS = 4
_I_CHUNKS = 32


def _copy_body(n_u, n_i, u_ref, i_ref, u_out, i_out, sem_u, sem_i):
    u_rows = u_ref.shape[0] // n_u
    i_rows = i_ref.shape[0] // n_i
    for k in range(n_u):
        pltpu.make_async_copy(
            u_ref.at[pl.ds(k * u_rows, u_rows), :],
            u_out.at[pl.ds(k * u_rows, u_rows), :],
            sem_u.at[k],
        ).start()
    for k in range(n_i):
        pltpu.make_async_copy(
            i_ref.at[pl.ds(k * i_rows, i_rows), :],
            i_out.at[pl.ds(k * i_rows, i_rows), :],
            sem_i.at[k],
        ).start()
    for k in range(n_u):
        pltpu.make_async_copy(
            u_ref.at[pl.ds(k * u_rows, u_rows), :],
            u_out.at[pl.ds(k * u_rows, u_rows), :],
            sem_u.at[k],
        ).wait()
    for k in range(n_i):
        pltpu.make_async_copy(
            i_ref.at[pl.ds(k * i_rows, i_rows), :],
            i_out.at[pl.ds(k * i_rows, i_rows), :],
            sem_i.at[k],
        ).wait()


def kernel(embedding_user, embedding_item):
    return pl.pallas_call(
        functools.partial(_copy_body, _U_CHUNKS, _I_CHUNKS),
        in_specs=[
            pl.BlockSpec(memory_space=pl.ANY),
            pl.BlockSpec(memory_space=pl.ANY),
        ],
        out_specs=[
            pl.BlockSpec(memory_space=pl.ANY),
            pl.BlockSpec(memory_space=pl.ANY),
        ],
        out_shape=[
            jax.ShapeDtypeStruct(embedding_user.shape, embedding_user.dtype),
            jax.ShapeDtypeStruct(embedding_item.shape, embedding_item.dtype),
        ],
        scratch_shapes=[
            pltpu.SemaphoreType.DMA((_U_CHUNKS,)),
            pltpu.SemaphoreType.DMA((_I_CHUNKS,)),
        ],
    )(embedding_user, embedding_item)


# speedup vs baseline: 1.8129x; 1.8129x over previous
import functools

import jax
import jax.numpy as jnp
from jax.experimental import pallas as pl
from jax.experimental.pallas import tpu as pltpu

_U_CHUNKS = 2
_I_CHUNKS = 20


def _copy_body(n_u, n_i, u_ref, i_ref, u_out, i_out, sem_u, sem_i):
    u_rows = u_ref.shape[0] // n_u
    i_rows = i_ref.shape[0] // n_i
    for k in range(n_u):
        pltpu.make_async_copy(
            u_ref.at[pl.ds(k * u_rows, u_rows), :],
            u_out.at[pl.ds(k * u_rows, u_rows), :],
            sem_u.at[k],
        ).start()
    for k in range(n_i):
        pltpu.make_async_copy(
            i_ref.at[pl.ds(k * i_rows, i_rows), :],
            i_out.at[pl.ds(k * i_rows, i_rows), :],
            sem_i.at[k],
        ).start()
    for k in range(n_u):
        pltpu.make_async_copy(
            u_ref.at[pl.ds(k * u_rows, u_rows), :],
            u_out.at[pl.ds(k * u_rows, u_rows), :],
            sem_u.at[k],
        ).wait()
    for k in range(n_i):
        pltpu.make_async_copy(
            i_ref.at[pl.ds(k * i_rows, i_rows), :],
            i_out.at[pl.ds(k * i_rows, i_rows), :],
            sem_i.at[k],
        ).wait()


def kernel(embedding_user, embedding_item):
    u_shape, i_shape = embedding_user.shape, embedding_item.shape
    u2 = embedding_user.reshape(-1, 128)
    i2 = embedding_item.reshape(-1, 128)
    u_out, i_out = pl.pallas_call(
        functools.partial(_copy_body, _U_CHUNKS, _I_CHUNKS),
        in_specs=[
            pl.BlockSpec(memory_space=pl.ANY),
            pl.BlockSpec(memory_space=pl.ANY),
        ],
        out_specs=[
            pl.BlockSpec(memory_space=pl.ANY),
            pl.BlockSpec(memory_space=pl.ANY),
        ],
        out_shape=[
            jax.ShapeDtypeStruct(u2.shape, u2.dtype),
            jax.ShapeDtypeStruct(i2.shape, i2.dtype),
        ],
        scratch_shapes=[
            pltpu.SemaphoreType.DMA((_U_CHUNKS,)),
            pltpu.SemaphoreType.DMA((_I_CHUNKS,)),
        ],
    )(u2, i2)
    return (u_out.reshape(u_shape), i_out.reshape(i_shape))


# speedup vs baseline: 12.2481x; 6.7559x over previous
import jax
import jax.numpy as jnp
from jax.experimental import pallas as pl
from jax.experimental.pallas import tpu as pltpu


def _copy_block(x_ref, o_ref):
    o_ref[...] = x_ref[...]


def _pipelined_copy(x, block_rows):
    rows = x.shape[0]
    assert rows % block_rows == 0
    return pl.pallas_call(
        _copy_block,
        grid=(rows // block_rows,),
        in_specs=[pl.BlockSpec((block_rows, 128), lambda i: (i, 0))],
        out_specs=pl.BlockSpec((block_rows, 128), lambda i: (i, 0)),
        out_shape=jax.ShapeDtypeStruct(x.shape, x.dtype),
    )(x)


def kernel(embedding_user, embedding_item):
    u_shape, i_shape = embedding_user.shape, embedding_item.shape
    u2 = embedding_user.reshape(-1, 128)
    i2 = embedding_item.reshape(-1, 128)
    u_out = _pipelined_copy(u2, 5000)
    i_out = _pipelined_copy(i2, 5000)
    return (u_out.reshape(u_shape), i_out.reshape(i_shape))
